# baseline (speedup 1.0000x reference)
import functools

import jax
import jax.numpy as jnp
from jax.experimental import pallas as pl
from jax.experimental.pallas import tpu as pltpu

_NH = 8
_C = 512


def _mm_kernel(*refs, nk, mode):
    x_ref, w_ref = refs[0], refs[1]
    o_ref, acc_ref = refs[-2], refs[-1]
    k = pl.program_id(2)

    @pl.when(k == 0)
    def _():
        acc_ref[...] = jnp.zeros_like(acc_ref)

    acc_ref[...] += jax.lax.dot_general(
        x_ref[...], w_ref[...], (((1,), (1,)), ((), ())),
        preferred_element_type=jnp.float32)

    @pl.when(k == nk - 1)
    def _():
        i = 2
        acc = acc_ref[...]
        if mode in ("bias", "gate", "res"):
            acc = acc + refs[i][...]
            i += 1
        if mode == "gate":
            z = jax.nn.sigmoid(acc)
            acc = z * refs[i][...] + (1.0 - z) * refs[i + 1][...]
            i += 2
        elif mode == "res":
            acc = acc + refs[i][...]
            i += 1
        o_ref[...] = acc


def _mm(x, w, b=None, gate=None, res=None, bm=256, bn=512, bk=512):
    m, kdim = x.shape
    n = w.shape[0]
    nk = kdim // bk
    grid = (m // bm, n // bn, nk)
    mode = "plain" if b is None else "bias"
    inputs = [x, w]
    in_specs = [
        pl.BlockSpec((bm, bk), lambda i, j, k: (i, k)),
        pl.BlockSpec((bn, bk), lambda i, j, k: (j, k)),
    ]
    if b is not None:
        inputs.append(b.reshape(1, n))
        in_specs.append(pl.BlockSpec((1, bn), lambda i, j, k: (0, j)))
    if gate is not None:
        mode = "gate"
        inputs.extend(gate)
        in_specs.extend([pl.BlockSpec((bm, bn), lambda i, j, k: (i, j))] * 2)
    if res is not None:
        mode = "res"
        inputs.append(res)
        in_specs.append(pl.BlockSpec((bm, bn), lambda i, j, k: (i, j)))
    return pl.pallas_call(
        functools.partial(_mm_kernel, nk=nk, mode=mode),
        grid=grid,
        in_specs=in_specs,
        out_specs=pl.BlockSpec((bm, bn), lambda i, j, k: (i, j)),
        out_shape=jax.ShapeDtypeStruct((m, n), jnp.float32),
        scratch_shapes=[pltpu.VMEM((bm, bn), jnp.float32)],
        compiler_params=pltpu.CompilerParams(
            dimension_semantics=("parallel", "parallel", "arbitrary")),
    )(*inputs)


def _elu_bias_kernel(x_ref, b_ref, o_ref):
    v = x_ref[...] + b_ref[...]
    o_ref[...] = jnp.where(v > 0, v, jnp.exp(jnp.minimum(v, 0.0)) - 1.0)


def _elu_bias(x, b, bm=256, bn=512):
    m, n = x.shape
    return pl.pallas_call(
        _elu_bias_kernel,
        grid=(m // bm, n // bn),
        in_specs=[
            pl.BlockSpec((bm, bn), lambda i, j: (i, j)),
            pl.BlockSpec((1, bn), lambda i, j: (0, j)),
        ],
        out_specs=pl.BlockSpec((bm, bn), lambda i, j: (i, j)),
        out_shape=jax.ShapeDtypeStruct((m, n), jnp.float32),
    )(x, b.reshape(1, n))


def _score_w(att, row0, nrows=128):
    w = (jnp.eye(_NH, dtype=jnp.float32)[:, :, None] * att[:, None, :])
    w = w.reshape(_NH, _NH * _C)
    return jnp.zeros((nrows, _NH * _C), jnp.float32).at[row0:row0 + _NH].set(w)


def _pad_rows(x, m):
    return jnp.pad(x, ((0, m - x.shape[0]), (0, 0)))


def _self_loops(src, dst, num_nodes, drop):
    keep = src != dst
    dst = jnp.where(keep, dst, jnp.asarray(drop, dst.dtype))
    loop = jnp.arange(num_nodes, dtype=src.dtype)
    return jnp.concatenate([src, loop]), jnp.concatenate([dst, loop])


def _gat_aggregate(sc_src, sc_dst, x_src3, src, dst, n_dst):
    dc = jnp.clip(dst, 0, n_dst - 1)
    a = sc_src[src] + sc_dst[dc]
    a = jnp.where(a >= 0, a, 0.2 * a)
    amax = jax.ops.segment_max(a, dst, num_segments=n_dst)
    amax = jnp.where(jnp.isfinite(amax), amax, 0.0)
    ex = jnp.exp(a - amax[dc])
    den = jax.ops.segment_sum(ex, dst, num_segments=n_dst)
    alpha = ex / (den[dc] + 1e-16)
    msg = x_src3[src] * alpha[:, :, None]
    out = jax.ops.segment_sum(msg, dst, num_segments=n_dst)
    return out.reshape(n_dst, -1)


def kernel(HS, Hs, s2S, S2S, Wsrc_s, Wdst_s, att_src_s, att_dst_s, bias_s,
           W_S, att_src_S, att_dst_S, bias_S, Wf, bf, W1, b1, W2, b2):
    n_S = HS.shape[0]
    n_s = Hs.shape[0]
    hc = _NH * _C
    mS = ((n_S + 255) // 256) * 256
    ms = ((n_s + 255) // 256) * 256
    HSp = _pad_rows(HS, mS)
    Hsp = _pad_rows(Hs, ms)

    Xs = _mm(Hsp, Wsrc_s, bk=128)
    Xd = _mm(HSp, Wdst_s)
    XS = _mm(HSp, W_S)

    w_sc_s = _score_w(att_src_s, 0) + _score_w(att_dst_s, _NH)
    w_sc_S = _score_w(att_src_S, 0) + _score_w(att_dst_S, _NH)
    sc_s_src = _mm(Xs, w_sc_s, bn=128)[:n_s, :_NH]
    sc_s_dst = _mm(Xd, w_sc_s, bn=128)[:n_S, _NH:2 * _NH]
    sc_S = _mm(XS, w_sc_S, bn=128)[:n_S]

    src1, dst1 = _self_loops(s2S[0], s2S[1], min(n_s, n_S), n_S)
    src2, dst2 = _self_loops(S2S[0], S2S[1], n_S, n_S)

    Us_raw = _gat_aggregate(sc_s_src, sc_s_dst,
                            Xs[:n_s].reshape(n_s, _NH, _C), src1, dst1, n_S)
    US_raw = _gat_aggregate(sc_S[:, :_NH], sc_S[:, _NH:2 * _NH],
                            XS[:n_S].reshape(n_S, _NH, _C), src2, dst2, n_S)

    Usp = _elu_bias(_pad_rows(Us_raw, mS), bias_s)
    USp = _elu_bias(_pad_rows(US_raw, mS), bias_S)

    Up = _mm(jnp.concatenate([USp, Usp], axis=1), Wf, b=bf, gate=(USp, Usp))
    U1 = _mm(Up, W1, b=b1)
    out = _mm(U1, W2, b=b2, res=HSp)
    return out[:n_S]


# speedup vs baseline: 1.5675x; 1.5675x over previous
import functools

import jax
import jax.numpy as jnp
from jax.experimental import pallas as pl
from jax.experimental.pallas import tpu as pltpu

_NH = 8
_C = 512


def _mm_kernel(*refs, nk, mode):
    x_ref, w_ref = refs[0], refs[1]
    o_ref, acc_ref = refs[-2], refs[-1]
    k = pl.program_id(2)

    @pl.when(k == 0)
    def _():
        acc_ref[...] = jnp.zeros_like(acc_ref)

    acc_ref[...] += jax.lax.dot_general(
        x_ref[...], w_ref[...], (((1,), (1,)), ((), ())),
        preferred_element_type=jnp.float32)

    @pl.when(k == nk - 1)
    def _():
        i = 2
        acc = acc_ref[...]
        if mode in ("bias", "gate", "res"):
            acc = acc + refs[i][...]
            i += 1
        if mode == "gate":
            z = jax.nn.sigmoid(acc)
            u0 = refs[i][...].astype(jnp.float32)
            u1 = refs[i + 1][...].astype(jnp.float32)
            acc = z * u0 + (1.0 - z) * u1
            i += 2
        elif mode == "res":
            acc = acc + refs[i][...]
            i += 1
        o_ref[...] = acc.astype(o_ref.dtype)


def _mm(x, w, b=None, gate=None, res=None, bm=1024, bn=1024, bk=512,
        out_dtype=jnp.float32):
    m, kdim = x.shape
    n = w.shape[0]
    bm = min(bm, m)
    bn = min(bn, n)
    nk = kdim // bk
    x = x.astype(jnp.bfloat16)
    w = w.astype(jnp.bfloat16)
    grid = (m // bm, n // bn, nk)
    mode = "plain" if b is None else "bias"
    inputs = [x, w]
    in_specs = [
        pl.BlockSpec((bm, bk), lambda i, j, k: (i, k)),
        pl.BlockSpec((bn, bk), lambda i, j, k: (j, k)),
    ]
    if b is not None:
        inputs.append(b.reshape(1, n))
        in_specs.append(pl.BlockSpec((1, bn), lambda i, j, k: (0, j)))
    if gate is not None:
        mode = "gate"
        inputs.extend(gate)
        in_specs.extend([pl.BlockSpec((bm, bn), lambda i, j, k: (i, j))] * 2)
    if res is not None:
        mode = "res"
        inputs.append(res)
        in_specs.append(pl.BlockSpec((bm, bn), lambda i, j, k: (i, j)))
    return pl.pallas_call(
        functools.partial(_mm_kernel, nk=nk, mode=mode),
        grid=grid,
        in_specs=in_specs,
        out_specs=pl.BlockSpec((bm, bn), lambda i, j, k: (i, j)),
        out_shape=jax.ShapeDtypeStruct((m, n), out_dtype),
        scratch_shapes=[pltpu.VMEM((bm, bn), jnp.float32)],
        compiler_params=pltpu.CompilerParams(
            dimension_semantics=("parallel", "parallel", "arbitrary")),
    )(*inputs)


def _elu_bias_kernel(x_ref, b_ref, o_ref):
    v = x_ref[...] + b_ref[...]
    v = jnp.where(v > 0, v, jnp.exp(jnp.minimum(v, 0.0)) - 1.0)
    o_ref[...] = v.astype(o_ref.dtype)


def _elu_bias(x, b, bm=1024, bn=1024):
    m, n = x.shape
    bm = min(bm, m)
    bn = min(bn, n)
    return pl.pallas_call(
        _elu_bias_kernel,
        grid=(m // bm, n // bn),
        in_specs=[
            pl.BlockSpec((bm, bn), lambda i, j: (i, j)),
            pl.BlockSpec((1, bn), lambda i, j: (0, j)),
        ],
        out_specs=pl.BlockSpec((bm, bn), lambda i, j: (i, j)),
        out_shape=jax.ShapeDtypeStruct((m, n), jnp.bfloat16),
    )(x, b.reshape(1, n))


def _score_w(att, row0, nrows=128):
    w = (jnp.eye(_NH, dtype=jnp.float32)[:, :, None] * att[:, None, :])
    w = w.reshape(_NH, _NH * _C)
    return jnp.zeros((nrows, _NH * _C), jnp.float32).at[row0:row0 + _NH].set(w)


def _pad_rows(x, m):
    return jnp.pad(x, ((0, m - x.shape[0]), (0, 0)))


def _self_loops(src, dst, num_nodes, drop):
    keep = src != dst
    dst = jnp.where(keep, dst, jnp.asarray(drop, dst.dtype))
    loop = jnp.arange(num_nodes, dtype=src.dtype)
    return jnp.concatenate([src, loop]), jnp.concatenate([dst, loop])


def _gat_aggregate(sc_src, sc_dst, x_src3, src, dst, n_dst):
    dc = jnp.clip(dst, 0, n_dst - 1)
    a = sc_src[src] + sc_dst[dc]
    a = jnp.where(a >= 0, a, 0.2 * a)
    amax = jax.ops.segment_max(a, dst, num_segments=n_dst)
    amax = jnp.where(jnp.isfinite(amax), amax, 0.0)
    ex = jnp.exp(a - amax[dc])
    den = jax.ops.segment_sum(ex, dst, num_segments=n_dst)
    alpha = ex / (den[dc] + 1e-16)
    msg = x_src3[src] * alpha[:, :, None]
    out = jax.ops.segment_sum(msg, dst, num_segments=n_dst)
    return out.reshape(n_dst, -1)


def kernel(HS, Hs, s2S, S2S, Wsrc_s, Wdst_s, att_src_s, att_dst_s, bias_s,
           W_S, att_src_S, att_dst_S, bias_S, Wf, bf, W1, b1, W2, b2):
    n_S = HS.shape[0]
    n_s = Hs.shape[0]
    hc = _NH * _C
    mS = ((n_S + 1023) // 1024) * 1024
    ms = ((n_s + 1023) // 1024) * 1024
    HSp = _pad_rows(HS, mS)
    Hsp = _pad_rows(Hs, ms)

    Xs = _mm(Hsp, Wsrc_s, bk=128, out_dtype=jnp.bfloat16)
    XS = _mm(HSp, W_S, out_dtype=jnp.bfloat16)

    w_sc_s = _score_w(att_src_s, 0) + _score_w(att_dst_s, _NH)
    w_sc_S = _score_w(att_src_S, 0) + _score_w(att_dst_S, _NH)
    sc_s_src = _mm(Hsp, w_sc_s @ Wsrc_s, bn=128, bk=128)[:n_s, :_NH]
    sc_s_dst = _mm(HSp, w_sc_s @ Wdst_s, bn=128)[:n_S, _NH:2 * _NH]
    sc_S = _mm(HSp, w_sc_S @ W_S, bn=128)[:n_S]

    src1, dst1 = _self_loops(s2S[0], s2S[1], min(n_s, n_S), n_S)
    src2, dst2 = _self_loops(S2S[0], S2S[1], n_S, n_S)

    Us_raw = _gat_aggregate(sc_s_src, sc_s_dst,
                            Xs[:n_s].reshape(n_s, _NH, _C), src1, dst1, n_S)
    US_raw = _gat_aggregate(sc_S[:, :_NH], sc_S[:, _NH:2 * _NH],
                            XS[:n_S].reshape(n_S, _NH, _C), src2, dst2, n_S)

    Usp = _elu_bias(_pad_rows(Us_raw, mS), bias_s)
    USp = _elu_bias(_pad_rows(US_raw, mS), bias_S)

    Up = _mm(jnp.concatenate([USp, Usp], axis=1), Wf, b=bf, gate=(USp, Usp),
             out_dtype=jnp.bfloat16)
    U1 = _mm(Up, W1, b=b1, out_dtype=jnp.bfloat16)
    out = _mm(U1, W2, b=b2, res=HSp)
    return out[:n_S]


# speedup vs baseline: 1.5891x; 1.0138x over previous
import functools

import jax
import jax.numpy as jnp
from jax.experimental import pallas as pl
from jax.experimental.pallas import tpu as pltpu

_NH = 8
_C = 512


def _mm_kernel(*refs, nk, mode):
    x_ref, w_ref = refs[0], refs[1]
    o_ref, acc_ref = refs[-2], refs[-1]
    k = pl.program_id(2)

    @pl.when(k == 0)
    def _():
        acc_ref[...] = jnp.zeros_like(acc_ref)

    acc_ref[...] += jax.lax.dot_general(
        x_ref[...], w_ref[...], (((1,), (1,)), ((), ())),
        preferred_element_type=jnp.float32)

    @pl.when(k == nk - 1)
    def _():
        i = 2
        acc = acc_ref[...]
        if mode in ("bias", "gate", "res"):
            acc = acc + refs[i][...]
            i += 1
        if mode == "gate":
            z = jax.nn.sigmoid(acc)
            u0 = refs[i][...].astype(jnp.float32)
            u1 = refs[i + 1][...].astype(jnp.float32)
            acc = z * u0 + (1.0 - z) * u1
            i += 2
        elif mode == "res":
            acc = acc + refs[i][...]
            i += 1
        o_ref[...] = acc.astype(o_ref.dtype)


def _mm(x, w, b=None, gate=None, res=None, bm=1024, bn=1024, bk=512,
        out_dtype=jnp.float32):
    m, kdim = x.shape
    n = w.shape[0]
    bm = min(bm, m)
    bn = min(bn, n)
    nk = kdim // bk
    x = x.astype(jnp.bfloat16)
    w = w.astype(jnp.bfloat16)
    grid = (m // bm, n // bn, nk)
    mode = "plain" if b is None else "bias"
    inputs = [x, w]
    in_specs = [
        pl.BlockSpec((bm, bk), lambda i, j, k: (i, k)),
        pl.BlockSpec((bn, bk), lambda i, j, k: (j, k)),
    ]
    if b is not None:
        inputs.append(b.reshape(1, n))
        in_specs.append(pl.BlockSpec((1, bn), lambda i, j, k: (0, j)))
    if gate is not None:
        mode = "gate"
        inputs.extend(gate)
        in_specs.extend([pl.BlockSpec((bm, bn), lambda i, j, k: (i, j))] * 2)
    if res is not None:
        mode = "res"
        inputs.append(res)
        in_specs.append(pl.BlockSpec((bm, bn), lambda i, j, k: (i, j)))
    return pl.pallas_call(
        functools.partial(_mm_kernel, nk=nk, mode=mode),
        grid=grid,
        in_specs=in_specs,
        out_specs=pl.BlockSpec((bm, bn), lambda i, j, k: (i, j)),
        out_shape=jax.ShapeDtypeStruct((m, n), out_dtype),
        scratch_shapes=[pltpu.VMEM((bm, bn), jnp.float32)],
        compiler_params=pltpu.CompilerParams(
            dimension_semantics=("parallel", "parallel", "arbitrary")),
    )(*inputs)


def _elu_bias_kernel(x_ref, b_ref, o_ref):
    v = x_ref[...] + b_ref[...]
    v = jnp.where(v > 0, v, jnp.exp(jnp.minimum(v, 0.0)) - 1.0)
    o_ref[...] = v.astype(o_ref.dtype)


def _elu_bias(x, b, bm=1024, bn=1024):
    m, n = x.shape
    bm = min(bm, m)
    bn = min(bn, n)
    return pl.pallas_call(
        _elu_bias_kernel,
        grid=(m // bm, n // bn),
        in_specs=[
            pl.BlockSpec((bm, bn), lambda i, j: (i, j)),
            pl.BlockSpec((1, bn), lambda i, j: (0, j)),
        ],
        out_specs=pl.BlockSpec((bm, bn), lambda i, j: (i, j)),
        out_shape=jax.ShapeDtypeStruct((m, n), jnp.bfloat16),
    )(x, b.reshape(1, n))


def _score_w(att, row0, nrows=128):
    w = (jnp.eye(_NH, dtype=jnp.float32)[:, :, None] * att[:, None, :])
    w = w.reshape(_NH, _NH * _C)
    return jnp.zeros((nrows, _NH * _C), jnp.float32).at[row0:row0 + _NH].set(w)


def _pad_rows(x, m):
    return jnp.pad(x, ((0, m - x.shape[0]), (0, 0)))


def _self_loops(src, dst, num_nodes, drop):
    keep = src != dst
    dst = jnp.where(keep, dst, jnp.asarray(drop, dst.dtype))
    loop = jnp.arange(num_nodes, dtype=src.dtype)
    return jnp.concatenate([src, loop]), jnp.concatenate([dst, loop])


def _gat_aggregate(sc_src, sc_dst, x_src3, src, dst, n_dst):
    dc = jnp.clip(dst, 0, n_dst - 1)
    a = sc_src[src] + sc_dst[dc]
    a = jnp.where(a >= 0, a, 0.2 * a)
    amax = jax.ops.segment_max(a, dst, num_segments=n_dst)
    amax = jnp.where(jnp.isfinite(amax), amax, 0.0)
    ex = jnp.exp(a - amax[dc])
    den = jax.ops.segment_sum(ex, dst, num_segments=n_dst)
    alpha = ex / (den[dc] + 1e-16)
    msg = x_src3[src] * alpha[:, :, None]
    out = jax.ops.segment_sum(msg, dst, num_segments=n_dst)
    return out.reshape(n_dst, -1)


def kernel(HS, Hs, s2S, S2S, Wsrc_s, Wdst_s, att_src_s, att_dst_s, bias_s,
           W_S, att_src_S, att_dst_S, bias_S, Wf, bf, W1, b1, W2, b2):
    n_S = HS.shape[0]
    n_s = Hs.shape[0]
    hc = _NH * _C
    mS = ((n_S + 1023) // 1024) * 1024
    ms = ((n_s + 1023) // 1024) * 1024
    HSp = _pad_rows(HS, mS)
    Hsp = _pad_rows(Hs, ms)

    Xs = _mm(Hsp, Wsrc_s, bk=128, out_dtype=jnp.bfloat16)
    XS = _mm(HSp, W_S, out_dtype=jnp.bfloat16)

    w_sc_s = _score_w(att_src_s, 0) + _score_w(att_dst_s, _NH)
    w_sc_S = _score_w(att_src_S, 0) + _score_w(att_dst_S, _NH)
    sc_s_src = _mm(Hsp, w_sc_s @ Wsrc_s, bn=128, bk=128)[:n_s, :_NH]
    sc_s_dst = _mm(HSp, w_sc_s @ Wdst_s, bn=128)[:n_S, _NH:2 * _NH]
    sc_S = _mm(HSp, w_sc_S @ W_S, bn=128)[:n_S]

    src1, dst1 = _self_loops(s2S[0], s2S[1], min(n_s, n_S), n_S)
    src2, dst2 = _self_loops(S2S[0], S2S[1], n_S, n_S)

    Us_raw = _gat_aggregate(sc_s_src, sc_s_dst,
                            Xs[:n_s].reshape(n_s, _NH, _C), src1, dst1, n_S)
    US_raw = _gat_aggregate(sc_S[:, :_NH], sc_S[:, _NH:2 * _NH],
                            XS[:n_S].reshape(n_S, _NH, _C), src2, dst2, n_S)

    def _elu(v):
        return jnp.where(v > 0, v, jnp.exp(jnp.minimum(v, 0.0)) - 1.0)

    Usp = _pad_rows(_elu(Us_raw + bias_s), mS).astype(jnp.bfloat16)
    USp = _pad_rows(_elu(US_raw + bias_S), mS).astype(jnp.bfloat16)

    Up = _mm(jnp.concatenate([USp, Usp], axis=1), Wf, b=bf, gate=(USp, Usp),
             out_dtype=jnp.bfloat16)
    U1 = _mm(Up, W1, b=b1, out_dtype=jnp.bfloat16)
    out = _mm(U1, W2, b=b2, res=HSp)
    return out[:n_S]


# speedup vs baseline: 1.6217x; 1.0205x over previous
import functools

import jax
import jax.numpy as jnp
from jax.experimental import pallas as pl
from jax.experimental.pallas import tpu as pltpu

_NH = 8
_C = 512


def _mm_kernel(*refs, nk, mode):
    x_ref, w_ref = refs[0], refs[1]
    o_ref, acc_ref = refs[-2], refs[-1]
    k = pl.program_id(2)

    @pl.when(k == 0)
    def _():
        acc_ref[...] = jnp.zeros_like(acc_ref)

    acc_ref[...] += jax.lax.dot_general(
        x_ref[...], w_ref[...], (((1,), (1,)), ((), ())),
        preferred_element_type=jnp.float32)

    @pl.when(k == nk - 1)
    def _():
        i = 2
        acc = acc_ref[...]
        if mode in ("bias", "gate", "res"):
            acc = acc + refs[i][...]
            i += 1
        if mode == "gate":
            z = jax.nn.sigmoid(acc)
            u0 = refs[i][...].astype(jnp.float32)
            u1 = refs[i + 1][...].astype(jnp.float32)
            acc = z * u0 + (1.0 - z) * u1
            i += 2
        elif mode == "res":
            acc = acc + refs[i][...]
            i += 1
        o_ref[...] = acc.astype(o_ref.dtype)


def _mm(x, w, b=None, gate=None, res=None, bm=2048, bn=1024, bk=512,
        out_dtype=jnp.float32):
    m, kdim = x.shape
    n = w.shape[0]
    bm = min(bm, m)
    bn = min(bn, n)
    nk = kdim // bk
    x = x.astype(jnp.bfloat16)
    w = w.astype(jnp.bfloat16)
    grid = (m // bm, n // bn, nk)
    mode = "plain" if b is None else "bias"
    inputs = [x, w]
    in_specs = [
        pl.BlockSpec((bm, bk), lambda i, j, k: (i, k)),
        pl.BlockSpec((bn, bk), lambda i, j, k: (j, k)),
    ]
    if b is not None:
        inputs.append(b.reshape(1, n))
        in_specs.append(pl.BlockSpec((1, bn), lambda i, j, k: (0, j)))
    if gate is not None:
        mode = "gate"
        inputs.extend(gate)
        in_specs.extend([pl.BlockSpec((bm, bn), lambda i, j, k: (i, j))] * 2)
    if res is not None:
        mode = "res"
        inputs.append(res)
        in_specs.append(pl.BlockSpec((bm, bn), lambda i, j, k: (i, j)))
    return pl.pallas_call(
        functools.partial(_mm_kernel, nk=nk, mode=mode),
        grid=grid,
        in_specs=in_specs,
        out_specs=pl.BlockSpec((bm, bn), lambda i, j, k: (i, j)),
        out_shape=jax.ShapeDtypeStruct((m, n), out_dtype),
        scratch_shapes=[pltpu.VMEM((bm, bn), jnp.float32)],
        compiler_params=pltpu.CompilerParams(
            dimension_semantics=("parallel", "parallel", "arbitrary")),
    )(*inputs)


def _score_w(att, row0, nrows=128):
    w = (jnp.eye(_NH, dtype=jnp.float32)[:, :, None] * att[:, None, :])
    w = w.reshape(_NH, _NH * _C)
    return jnp.zeros((nrows, _NH * _C), jnp.float32).at[row0:row0 + _NH].set(w)


def _pad_rows(x, m):
    return jnp.pad(x, ((0, m - x.shape[0]), (0, 0)))


def _self_loops(src, dst, num_nodes, drop):
    keep = src != dst
    dst = jnp.where(keep, dst, jnp.asarray(drop, dst.dtype))
    loop = jnp.arange(num_nodes, dtype=src.dtype)
    return jnp.concatenate([src, loop]), jnp.concatenate([dst, loop])


def _gat_aggregate(sc_src, sc_dst, x_src3, src, dst, n_dst):
    dc = jnp.clip(dst, 0, n_dst - 1)
    a = sc_src[src] + sc_dst[dc]
    a = jnp.where(a >= 0, a, 0.2 * a)
    amax = jax.ops.segment_max(a, dst, num_segments=n_dst)
    amax = jnp.where(jnp.isfinite(amax), amax, 0.0)
    ex = jnp.exp(a - amax[dc])
    den = jax.ops.segment_sum(ex, dst, num_segments=n_dst)
    alpha = ex / (den[dc] + 1e-16)
    msg = x_src3[src] * alpha[:, :, None]
    out = jax.ops.segment_sum(msg, dst, num_segments=n_dst)
    return out.reshape(n_dst, -1)


def kernel(HS, Hs, s2S, S2S, Wsrc_s, Wdst_s, att_src_s, att_dst_s, bias_s,
           W_S, att_src_S, att_dst_S, bias_S, Wf, bf, W1, b1, W2, b2):
    n_S = HS.shape[0]
    n_s = Hs.shape[0]
    mS = ((n_S + 2047) // 2048) * 2048
    ms = ((n_s + 2047) // 2048) * 2048
    HSp = _pad_rows(HS, mS)
    Hsp = _pad_rows(Hs, ms)

    Xs = _mm(Hsp, Wsrc_s, bk=128, out_dtype=jnp.bfloat16)
    XS = _mm(HSp, W_S, out_dtype=jnp.bfloat16)

    w_sc_s = _score_w(att_src_s, 0) + _score_w(att_dst_s, _NH)
    w_sc_S = _score_w(att_src_S, 0) + _score_w(att_dst_S, _NH)
    sc_s_src = _mm(Hsp, w_sc_s @ Wsrc_s, bn=128, bk=128)[:n_s, :_NH]
    sc_s_dst = _mm(HSp, w_sc_s @ Wdst_s, bn=128)[:n_S, _NH:2 * _NH]
    sc_S = _mm(HSp, w_sc_S @ W_S, bn=128)[:n_S]

    src1, dst1 = _self_loops(s2S[0], s2S[1], min(n_s, n_S), n_S)
    src2, dst2 = _self_loops(S2S[0], S2S[1], n_S, n_S)

    Us_raw = _gat_aggregate(sc_s_src, sc_s_dst,
                            Xs[:n_s].reshape(n_s, _NH, _C), src1, dst1, n_S)
    US_raw = _gat_aggregate(sc_S[:, :_NH], sc_S[:, _NH:2 * _NH],
                            XS[:n_S].reshape(n_S, _NH, _C), src2, dst2, n_S)

    def _elu(v):
        return jnp.where(v > 0, v, jnp.exp(jnp.minimum(v, 0.0)) - 1.0)

    Usp = _pad_rows(_elu(Us_raw + bias_s), mS).astype(jnp.bfloat16)
    USp = _pad_rows(_elu(US_raw + bias_S), mS).astype(jnp.bfloat16)

    Up = _mm(jnp.concatenate([USp, Usp], axis=1), Wf, b=bf, gate=(USp, Usp),
             out_dtype=jnp.bfloat16)
    U1 = _mm(Up, W1, b=b1, out_dtype=jnp.bfloat16)
    out = _mm(U1, W2, b=b2, res=HSp)
    return out[:n_S]
